# Initial kernel scaffold; baseline (speedup 1.0000x reference)
import jax
import jax.numpy as jnp
from jax.experimental import pallas as pl


def kernel(X, W):
    raise NotImplementedError("write your pallas kernel here")


# speedup vs baseline: 1.4586x; 1.4586x over previous
import functools

import jax
import jax.numpy as jnp
from jax import lax
from jax.experimental import pallas as pl
from jax.experimental.pallas import tpu as pltpu
from jax.experimental.pallas import tpu_sc as plsc

NC = 2
NS = 16
NW = NC * NS

B = 4096 * 200
D = 32
BPW = B // NW
CHUNK = 1024
NCHUNK = BPW // CHUNK


def _build():
  mesh = plsc.VectorSubcoreMesh(core_axis_name="c", subcore_axis_name="s")

  @functools.partial(
      pl.kernel,
      mesh=mesh,
      out_type=jax.ShapeDtypeStruct((B, D), jnp.float32),
      scratch_types=[
          pltpu.VMEM((CHUNK,), jnp.int32),
          pltpu.VMEM((CHUNK, D), jnp.float32),
          pltpu.SemaphoreType.DMA,
      ],
      compiler_params=pltpu.CompilerParams(use_tc_tiling_on_sc=False),
  )
  def lookup(table_hbm, idx_hbm, out_hbm, idx_v, rows_v, sem):
    wid = lax.axis_index("s") * NC + lax.axis_index("c")
    base = wid * BPW

    def step(i, carry):
      off = base + i * CHUNK
      pltpu.sync_copy(idx_hbm.at[pl.ds(off, CHUNK)], idx_v)
      pltpu.async_copy(table_hbm.at[idx_v], rows_v, sem).wait()
      pltpu.sync_copy(rows_v, out_hbm.at[pl.ds(off, CHUNK)])
      return carry

    lax.fori_loop(0, NCHUNK, step, 0)

  return lookup


_lookup = _build()


@jax.jit
def kernel(X, W):
  idx = X.reshape(-1)
  out = _lookup(W, idx)
  return out.reshape(X.shape + (W.shape[1],))


# speedup vs baseline: 1.5012x; 1.0292x over previous
import functools

import jax
import jax.numpy as jnp
from jax import lax
from jax.experimental import pallas as pl
from jax.experimental.pallas import tpu as pltpu
from jax.experimental.pallas import tpu_sc as plsc

NC = 2
NS = 16
NW = NC * NS

B = 4096 * 200
D = 32
BPW = B // NW
NBUF = 4
CHUNK = 800
NCHUNK = BPW // CHUNK
NRING = NCHUNK // NBUF


def _build():
  mesh = plsc.VectorSubcoreMesh(core_axis_name="c", subcore_axis_name="s")

  scratch = (
      [pltpu.VMEM((CHUNK,), jnp.int32) for _ in range(NBUF)]
      + [pltpu.VMEM((CHUNK, D), jnp.float32) for _ in range(NBUF)]
      + [pltpu.SemaphoreType.DMA for _ in range(3 * NBUF)]
  )

  @functools.partial(
      pl.kernel,
      mesh=mesh,
      out_type=jax.ShapeDtypeStruct((B, D), jnp.float32),
      scratch_types=scratch,
      compiler_params=pltpu.CompilerParams(use_tc_tiling_on_sc=False),
  )
  def lookup(table_hbm, idx_hbm, out_hbm, *refs):
    idx_v = refs[0:NBUF]
    rows_v = refs[NBUF:2 * NBUF]
    idx_sem = refs[2 * NBUF:3 * NBUF]
    g_sem = refs[3 * NBUF:4 * NBUF]
    st_sem = refs[4 * NBUF:5 * NBUF]

    wid = lax.axis_index("s") * NC + lax.axis_index("c")
    base = wid * BPW

    def idx_start(i, s):
      pltpu.async_copy(idx_hbm.at[pl.ds(base + i * CHUNK, CHUNK)],
                       idx_v[s], idx_sem[s])

    def idx_wait(s):
      pltpu.make_async_copy(idx_hbm.at[pl.ds(0, CHUNK)],
                            idx_v[s], idx_sem[s]).wait()

    def gather_start(s):
      pltpu.async_copy(table_hbm.at[idx_v[s]], rows_v[s], g_sem[s])

    def gather_wait(s):
      pltpu.make_async_copy(table_hbm.at[idx_v[s]],
                            rows_v[s], g_sem[s]).wait()

    def store_start(i, s):
      pltpu.async_copy(rows_v[s],
                       out_hbm.at[pl.ds(base + i * CHUNK, CHUNK)], st_sem[s])

    def store_wait(s):
      pltpu.make_async_copy(rows_v[s],
                            out_hbm.at[pl.ds(0, CHUNK)], st_sem[s]).wait()

    for s in range(NBUF):
      idx_start(s, s)

    def ring(g, carry):
      for b in range(NBUF):
        i = g * NBUF + b
        idx_wait(b)
        pl.when(g >= 1)(lambda: store_wait(b))
        gather_start(b)
        bp = (b - 1) % NBUF
        if b == 0:
          def retire_prev_ring():
            gather_wait(bp)
            store_start(i - 1, bp)
            idx_start(i + NBUF - 1, bp)
          pl.when(g >= 1)(retire_prev_ring)
        else:
          gather_wait(bp)
          store_start(i - 1, bp)
          pl.when(g < NRING - 1)(
              functools.partial(idx_start, i + NBUF - 1, bp))
      return carry

    lax.fori_loop(0, NRING, ring, 0)

    last = NCHUNK - 1
    s_last = last % NBUF
    gather_wait(s_last)
    store_start(last, s_last)
    for s in range(NBUF):
      store_wait(s)

  return lookup


_lookup = _build()


@jax.jit
def kernel(X, W):
  idx = X.reshape(-1)
  out = _lookup(W, idx)
  return out.reshape(X.shape + (W.shape[1],))
